# baseline (speedup 1.0000x reference)
import functools

import jax
import jax.numpy as jnp
from jax import lax
from jax.experimental import pallas as pl
from jax.experimental.pallas import tpu as pltpu
from jax.experimental.pallas import tpu_sc as plsc

_NOISE = 0.5

_B, _H, _W, _C = 8, 224, 224, 192
_ROWS = _B * _H * _W
_NW = 32
_PER_W = _ROWS // _NW
_CHUNK = 128
_NCH = _PER_W // _CHUNK
_NBUF = 4


def _flat_indices():
    ii, jj = jnp.meshgrid(jnp.arange(_H, dtype=jnp.float32),
                          jnp.arange(_W, dtype=jnp.float32), indexing='ij')
    k1, k2 = jax.random.split(jax.random.key(123))
    n_i = jax.random.normal(k1, (_B, _H, _W, 1), dtype=jnp.float32)
    n_j = jax.random.normal(k2, (_B, _H, _W, 1), dtype=jnp.float32)
    a_i = ii[None, :, :, None] + n_i * _NOISE
    a_j = jj[None, :, :, None] + n_j * _NOISE
    a = jnp.concatenate([a_i, a_j], axis=3)
    a = jnp.floor(a + 0.4999).astype(jnp.int32)
    a = jnp.where(a < 0, 0, a)
    s = jnp.array([_H - 1, _W - 1], dtype=jnp.int32)
    a = jnp.where(a > s, s, a)
    flat = a[..., 0] * _W + a[..., 1]
    base = jnp.arange(_B, dtype=jnp.int32) * (_H * _W)
    return (flat + base[:, None, None]).reshape(_ROWS)


def _sc_gather(table, idx):
    mesh = plsc.VectorSubcoreMesh(core_axis_name="c", subcore_axis_name="s")

    @functools.partial(
        pl.kernel,
        out_type=jax.ShapeDtypeStruct((_ROWS, _C), jnp.float32),
        mesh=mesh,
        scratch_types=[
            pltpu.VMEM((_NCH, _CHUNK), jnp.int32),
            pltpu.VMEM((_NBUF, _CHUNK, _C), jnp.float32),
            pltpu.SemaphoreType.DMA,
            pltpu.SemaphoreType.DMA,
        ],
        compiler_params=pltpu.CompilerParams(use_tc_tiling_on_sc=False),
    )
    def k(table_hbm, idx_hbm, out_hbm, idx_v, rows_v, gsem, ssem):
        wid = lax.axis_index("s") * 2 + lax.axis_index("c")
        base = wid * _PER_W
        pltpu.sync_copy(idx_hbm.at[wid], idx_v)

        for b in range(_NBUF):
            pltpu.async_copy(table_hbm.at[idx_v.at[b]], rows_v.at[b], gsem)

        def body(j, _):
            b = lax.rem(j, _NBUF)
            pltpu.make_async_copy(table_hbm.at[idx_v.at[b]], rows_v.at[b],
                                  gsem).wait()
            pltpu.async_copy(
                rows_v.at[b], out_hbm.at[pl.ds(base + j * _CHUNK, _CHUNK)], ssem)

            @pl.when(j + _NBUF < _NCH)
            def _():
                pltpu.make_async_copy(
                    rows_v.at[b],
                    out_hbm.at[pl.ds(base + j * _CHUNK, _CHUNK)], ssem).wait()
                pltpu.async_copy(table_hbm.at[idx_v.at[j + _NBUF]],
                                 rows_v.at[b], gsem)
            return 0

        lax.fori_loop(0, _NCH, body, 0)
        for b in range(_NBUF):
            pltpu.make_async_copy(
                rows_v.at[b],
                out_hbm.at[pl.ds(base + (_NCH - _NBUF + b) * _CHUNK, _CHUNK)],
                ssem).wait()

    return k(table, idx)


def kernel(image):
    idx = _flat_indices().reshape(_NW, _NCH, _CHUNK)
    table = image.reshape(_ROWS, _C)
    out = _sc_gather(table, idx)
    return out.reshape(_B, _H, _W, _C)


# speedup vs baseline: 1.0946x; 1.0946x over previous
import functools

import jax
import jax.numpy as jnp
import numpy as np
from jax import lax
from jax.experimental import pallas as pl
from jax.experimental.pallas import tpu as pltpu
from jax.experimental.pallas import tpu_sc as plsc

_NOISE = 0.5
_B, _H, _W, _C = 8, 224, 224, 192
_ROWS = _B * _H * _W
_NW = 32
_NSTRIP = 4
_SW = _W // _NSTRIP
_RING = 6
_LOADW = 72
_GW0 = (0, 6, 13, 19)
_PER_IMG = _H * _W


def _index_tables():
    ii, jj = jnp.meshgrid(jnp.arange(_H, dtype=jnp.float32),
                          jnp.arange(_W, dtype=jnp.float32), indexing='ij')
    k1, k2 = jax.random.split(jax.random.key(123))
    n_i = jax.random.normal(k1, (_B, _H, _W, 1), dtype=jnp.float32)
    n_j = jax.random.normal(k2, (_B, _H, _W, 1), dtype=jnp.float32)
    a_i = ii[None, :, :, None] + n_i * _NOISE
    a_j = jj[None, :, :, None] + n_j * _NOISE
    a = jnp.concatenate([a_i, a_j], axis=3)
    a = jnp.floor(a + 0.4999).astype(jnp.int32)
    a = jnp.where(a < 0, 0, a)
    s = jnp.array([_H - 1, _W - 1], dtype=jnp.int32)
    a = jnp.where(a > s, s, a)
    ai, aj = a[..., 0], a[..., 1]
    gw0 = jnp.repeat(jnp.array(_GW0, jnp.int32), _SW)
    p = (ai % _RING) * _LOADW + (aj - 8 * gw0[None, None, :])
    p = p.reshape(_B, _H, _NSTRIP, _SW).transpose(0, 2, 1, 3)
    p = p.reshape(_NW, _H, _SW)
    return jnp.pad(p, ((0, 0), (0, 0), (0, 128 - _SW)))


def _sc_scramble(table, pidx):
    mesh = plsc.VectorSubcoreMesh(core_axis_name="c", subcore_axis_name="s")

    @functools.partial(
        pl.kernel,
        out_type=jax.ShapeDtypeStruct((_ROWS, _C), jnp.float32),
        mesh=mesh,
        scratch_types=[
            pltpu.VMEM((_RING * _LOADW, _C), jnp.float32),
            pltpu.VMEM((32, _C), jnp.float32),
            pltpu.VMEM((24, _C), jnp.float32),
            pltpu.VMEM((2, 8, 128), jnp.int32),
            pltpu.SemaphoreType.DMA,
            pltpu.SemaphoreType.DMA,
            pltpu.SemaphoreType.DMA,
            pltpu.SemaphoreType.DMA,
        ],
        compiler_params=pltpu.CompilerParams(use_tc_tiling_on_sc=True),
    )
    def k(table_hbm, pidx_hbm, out_hbm, band, oa, ob, pv, lsem, asem, bsem,
          psem):
        wid = lax.axis_index("s") * 2 + lax.axis_index("c")
        b = wid // _NSTRIP
        q = wid % _NSTRIP
        gw0 = jnp.clip(7 * q - 1, 0, 19)
        in0 = b * _PER_IMG + 8 * gw0
        ob0 = b * _PER_IMG + _SW * q

        def load_band(r):
            pltpu.async_copy(table_hbm.at[pl.ds(in0 + r * _W, _LOADW)],
                             band.at[pl.ds(lax.rem(r, _RING) * _LOADW,
                                           _LOADW)], lsem)

        def wait_band():
            pltpu.make_async_copy(table_hbm.at[pl.ds(0, _LOADW)],
                                  band.at[pl.ds(0, _LOADW)], lsem).wait()

        def load_pidx(c):
            pltpu.async_copy(pidx_hbm.at[wid, pl.ds(c * 8, 8)],
                             pv.at[lax.rem(c, 2)], psem)

        def wait_pidx():
            pltpu.make_async_copy(pidx_hbm.at[0, pl.ds(0, 8)], pv.at[0],
                                  psem).wait()

        for r in range(3):
            load_band(r)
        load_pidx(0)
        for _ in range(3):
            wait_band()
        wait_pidx()

        def row_body(i, _):
            @pl.when(i + 3 <= _H - 1)
            def _():
                load_band(i + 3)

            @pl.when(jnp.logical_and(i >= 1, i + 2 <= _H - 1))
            def _():
                wait_band()

            c = i // 8

            @pl.when(lax.rem(i, 8) == 0)
            def _():
                @pl.when(i > 0)
                def _():
                    wait_pidx()

                @pl.when(c + 1 <= _H // 8 - 1)
                def _():
                    load_pidx(c + 1)

            cb = lax.rem(c, 2)
            r8 = lax.rem(i, 8)

            def fill(lo, hi, dst):
                for g in range(lo // 16, (hi + 15) // 16):
                    pvec = pv[cb, r8, pl.ds(16 * g, 16)]
                    for l in range(min(16, hi - 16 * g)):
                        u = 16 * g + l
                        p = pvec[l]
                        for t in range(_C // 16):
                            dst[u - lo, pl.ds(16 * t, 16)] = (
                                band[p, pl.ds(16 * t, 16)])

            @pl.when(i >= 1)
            def _():
                pltpu.make_async_copy(oa, out_hbm.at[pl.ds(0, 32)],
                                      asem).wait()

            fill(0, 32, oa)
            pltpu.async_copy(oa, out_hbm.at[pl.ds(ob0 + i * _W, 32)], asem)

            @pl.when(i >= 1)
            def _():
                pltpu.make_async_copy(ob, out_hbm.at[pl.ds(0, 24)],
                                      bsem).wait()

            fill(32, 56, ob)
            pltpu.async_copy(ob, out_hbm.at[pl.ds(ob0 + i * _W + 32, 24)],
                             bsem)
            return 0

        lax.fori_loop(0, _H, row_body, 0)
        pltpu.make_async_copy(oa, out_hbm.at[pl.ds(0, 32)], asem).wait()
        pltpu.make_async_copy(ob, out_hbm.at[pl.ds(0, 24)], bsem).wait()

    return k(table, pidx)


def kernel(image):
    table = image.reshape(_ROWS, _C)
    out = _sc_scramble(table, _index_tables())
    return out.reshape(_B, _H, _W, _C)


# speedup vs baseline: 2.5917x; 2.3678x over previous
import functools

import jax
import jax.numpy as jnp
import numpy as np
from jax import lax
from jax.experimental import pallas as pl
from jax.experimental.pallas import tpu as pltpu
from jax.experimental.pallas import tpu_sc as plsc

_NOISE = 0.5
_B, _H, _W, _C = 8, 224, 224, 192
_ROWS = _B * _H * _W
_NW = 32
_NSTRIP = 4
_SW = _W // _NSTRIP
_RING = 6
_LOADW = 72
_GW0 = (0, 6, 13, 19)
_PER_IMG = _H * _W


def _index_tables():
    ii, jj = jnp.meshgrid(jnp.arange(_H, dtype=jnp.float32),
                          jnp.arange(_W, dtype=jnp.float32), indexing='ij')
    k1, k2 = jax.random.split(jax.random.key(123))
    n_i = jax.random.normal(k1, (_B, _H, _W, 1), dtype=jnp.float32)
    n_j = jax.random.normal(k2, (_B, _H, _W, 1), dtype=jnp.float32)
    a_i = ii[None, :, :, None] + n_i * _NOISE
    a_j = jj[None, :, :, None] + n_j * _NOISE
    a = jnp.concatenate([a_i, a_j], axis=3)
    a = jnp.floor(a + 0.4999).astype(jnp.int32)
    a = jnp.where(a < 0, 0, a)
    s = jnp.array([_H - 1, _W - 1], dtype=jnp.int32)
    a = jnp.where(a > s, s, a)
    ai, aj = a[..., 0], a[..., 1]
    gw0 = jnp.repeat(jnp.array(_GW0, jnp.int32), _SW)
    p = (ai % _RING) * _LOADW + (aj - 8 * gw0[None, None, :])
    p = p.reshape(_B, _H, _NSTRIP, _SW).transpose(0, 2, 1, 3)
    p = p.reshape(_NW, _H, _SW)
    return jnp.pad(p, ((0, 0), (0, 0), (0, 128 - _SW)))


def _sc_scramble(table, pidx):
    mesh = plsc.VectorSubcoreMesh(core_axis_name="c", subcore_axis_name="s")

    @functools.partial(
        pl.kernel,
        out_type=jax.ShapeDtypeStruct((_B, _H, _W, _C), jnp.float32),
        mesh=mesh,
        scratch_types=[
            pltpu.VMEM((_RING * _LOADW, _C), jnp.float32),
            pltpu.VMEM((32, _C), jnp.float32),
            pltpu.VMEM((24, _C), jnp.float32),
            pltpu.VMEM((2, 8, 128), jnp.int32),
            pltpu.SemaphoreType.DMA,
            pltpu.SemaphoreType.DMA,
            pltpu.SemaphoreType.DMA,
            pltpu.SemaphoreType.DMA,
        ],
        compiler_params=pltpu.CompilerParams(use_tc_tiling_on_sc=True),
    )
    def k(table_hbm, pidx_hbm, out_hbm, band, oa, ob, pv, lsem, asem, bsem,
          psem):
        wid = lax.axis_index("s") * 2 + lax.axis_index("c")
        b = wid // _NSTRIP
        q = wid % _NSTRIP
        gw0 = jnp.clip(7 * q - 1, 0, 19)
        w0 = 8 * gw0
        oc0 = _SW * q

        def load_band(r):
            pltpu.async_copy(table_hbm.at[b, r, pl.ds(w0, _LOADW)],
                             band.at[pl.ds(lax.rem(r, _RING) * _LOADW,
                                           _LOADW)], lsem)

        def wait_band():
            pltpu.make_async_copy(table_hbm.at[0, 0, pl.ds(0, _LOADW)],
                                  band.at[pl.ds(0, _LOADW)], lsem).wait()

        def load_pidx(c):
            pltpu.async_copy(pidx_hbm.at[wid, pl.ds(c * 8, 8)],
                             pv.at[lax.rem(c, 2)], psem)

        def wait_pidx():
            pltpu.make_async_copy(pidx_hbm.at[0, pl.ds(0, 8)], pv.at[0],
                                  psem).wait()

        for r in range(3):
            load_band(r)
        load_pidx(0)
        for _ in range(3):
            wait_band()
        wait_pidx()

        def row_body(i, _):
            @pl.when(i + 3 <= _H - 1)
            def _():
                load_band(i + 3)

            @pl.when(jnp.logical_and(i >= 1, i + 2 <= _H - 1))
            def _():
                wait_band()

            c = i // 8

            @pl.when(lax.rem(i, 8) == 0)
            def _():
                @pl.when(i > 0)
                def _():
                    wait_pidx()

                @pl.when(c + 1 <= _H // 8 - 1)
                def _():
                    load_pidx(c + 1)

            cb = lax.rem(c, 2)
            r8 = lax.rem(i, 8)

            def fill(lo, hi, dst):
                for g in range(lo // 16, (hi + 15) // 16):
                    pvec = pv[cb, r8, pl.ds(16 * g, 16)]
                    for l in range(min(16, hi - 16 * g)):
                        u = 16 * g + l
                        p = pvec[l]
                        for t in range(_C // 16):
                            dst[u - lo, pl.ds(16 * t, 16)] = (
                                band[p, pl.ds(16 * t, 16)])

            @pl.when(i >= 1)
            def _():
                pltpu.make_async_copy(oa, out_hbm.at[0, 0, pl.ds(0, 32)],
                                      asem).wait()

            fill(0, 32, oa)
            pltpu.async_copy(oa, out_hbm.at[b, i, pl.ds(oc0, 32)], asem)

            @pl.when(i >= 1)
            def _():
                pltpu.make_async_copy(ob, out_hbm.at[0, 0, pl.ds(0, 24)],
                                      bsem).wait()

            fill(32, 56, ob)
            pltpu.async_copy(ob, out_hbm.at[b, i, pl.ds(oc0 + 32, 24)],
                             bsem)
            return 0

        lax.fori_loop(0, _H, row_body, 0)
        pltpu.make_async_copy(oa, out_hbm.at[0, 0, pl.ds(0, 32)], asem).wait()
        pltpu.make_async_copy(ob, out_hbm.at[0, 0, pl.ds(0, 24)], bsem).wait()

    return k(table, pidx)


def kernel(image):
    return _sc_scramble(image, _index_tables())


# speedup vs baseline: 3.7310x; 1.4396x over previous
import functools

import jax
import jax.numpy as jnp
from jax import lax
from jax.experimental import pallas as pl
from jax.experimental.pallas import tpu as pltpu
from jax.experimental.pallas import tpu_sc as plsc

_NOISE = 0.5
_B, _H, _W, _C = 8, 224, 224, 192
_NW = 32
_NSLAB = 4
_SC = _C // _NSLAB
_RING = 6
_NG = _W // 16


def _index_tables():
    ii, jj = jnp.meshgrid(jnp.arange(_H, dtype=jnp.float32),
                          jnp.arange(_W, dtype=jnp.float32), indexing='ij')
    k1, k2 = jax.random.split(jax.random.key(123))
    n_i = jax.random.normal(k1, (_B, _H, _W, 1), dtype=jnp.float32)
    n_j = jax.random.normal(k2, (_B, _H, _W, 1), dtype=jnp.float32)
    a_i = ii[None, :, :, None] + n_i * _NOISE
    a_j = jj[None, :, :, None] + n_j * _NOISE
    a = jnp.concatenate([a_i, a_j], axis=3)
    a = jnp.floor(a + 0.4999).astype(jnp.int32)
    a = jnp.where(a < 0, 0, a)
    s = jnp.array([_H - 1, _W - 1], dtype=jnp.int32)
    a = jnp.where(a > s, s, a)
    ai, aj = a[..., 0], a[..., 1]
    return ((ai % _RING) * _SC << 8) | aj


def _sc_scramble(table, pidx):
    mesh = plsc.VectorSubcoreMesh(core_axis_name="c", subcore_axis_name="s")

    @functools.partial(
        pl.kernel,
        out_type=jax.ShapeDtypeStruct((_B, _H, _C, _W), jnp.float32),
        mesh=mesh,
        scratch_types=[
            pltpu.VMEM((_RING * _SC, _W), jnp.float32),
            pltpu.VMEM((2, _SC, _W), jnp.float32),
            pltpu.VMEM((2, 8, _W), jnp.int32),
            pltpu.SemaphoreType.DMA,
            pltpu.SemaphoreType.DMA,
            pltpu.SemaphoreType.DMA,
        ],
        compiler_params=pltpu.CompilerParams(use_tc_tiling_on_sc=True,
                                             needs_layout_passes=False),
    )
    def k(table_hbm, pidx_hbm, out_hbm, band, obuf, pv, lsem, ssem, psem):
        wid = lax.axis_index("s") * 2 + lax.axis_index("c")
        b = wid // _NSLAB
        c0 = (wid % _NSLAB) * _SC

        def load_band(r):
            pltpu.async_copy(table_hbm.at[b, r, pl.ds(c0, _SC)],
                             band.at[pl.ds(lax.rem(r, _RING) * _SC, _SC)],
                             lsem)

        def wait_band():
            pltpu.make_async_copy(table_hbm.at[0, 0, pl.ds(0, _SC)],
                                  band.at[pl.ds(0, _SC)], lsem).wait()

        def load_pidx(c):
            pltpu.async_copy(pidx_hbm.at[b, pl.ds(c * 8, 8)],
                             pv.at[lax.rem(c, 2)], psem)

        def wait_pidx():
            pltpu.make_async_copy(pidx_hbm.at[0, pl.ds(0, 8)], pv.at[0],
                                  psem).wait()

        def wait_store():
            pltpu.make_async_copy(obuf.at[0], out_hbm.at[0, 0, pl.ds(0, _SC)],
                                  ssem).wait()

        for r in range(3):
            load_band(r)
        load_pidx(0)
        for _ in range(3):
            wait_band()
        wait_pidx()

        def row_body(i, _):
            @pl.when(i + 3 <= _H - 1)
            def _():
                load_band(i + 3)

            @pl.when(jnp.logical_and(i >= 1, i + 2 <= _H - 1))
            def _():
                wait_band()

            c = i // 8

            @pl.when(lax.rem(i, 8) == 0)
            def _():
                @pl.when(i > 0)
                def _():
                    wait_pidx()

                @pl.when(c + 1 <= _H // 8 - 1)
                def _():
                    load_pidx(c + 1)

            cb = lax.rem(c, 2)
            r8 = lax.rem(i, 8)
            d = lax.rem(i, 2)

            @pl.when(i >= 2)
            def _():
                wait_store()

            for g in range(_NG):
                pvec = pv[cb, r8, pl.ds(16 * g, 16)]
                rb = lax.shift_right_logical(pvec, 8)
                cv = jnp.bitwise_and(pvec, 255)

                def ch_body(ch, carry):
                    row = rb + ch
                    val = plsc.load_gather(band, [row, cv])
                    obuf[d, ch, pl.ds(16 * g, 16)] = val
                    return carry

                lax.fori_loop(0, _SC, ch_body, 0, unroll=8)

            pltpu.async_copy(obuf.at[d], out_hbm.at[b, i, pl.ds(c0, _SC)],
                             ssem)
            return 0

        lax.fori_loop(0, _H, row_body, 0)
        wait_store()
        wait_store()

    return k(table, pidx)


def kernel(image):
    table = image.transpose(0, 1, 3, 2)
    out = _sc_scramble(table, _index_tables())
    return out.transpose(0, 1, 3, 2)


# speedup vs baseline: 4.4371x; 1.1892x over previous
import functools

import jax
import jax.numpy as jnp
from jax import lax
from jax.experimental import pallas as pl
from jax.experimental.pallas import tpu as pltpu
from jax.experimental.pallas import tpu_sc as plsc

_NOISE = 0.5
_B, _H, _W, _C = 8, 224, 224, 192
_NW = 32
_NSLAB = 4
_SC = _C // _NSLAB
_RING = 6
_NG = _W // 16


def _index_tables():
    ii, jj = jnp.meshgrid(jnp.arange(_H, dtype=jnp.float32),
                          jnp.arange(_W, dtype=jnp.float32), indexing='ij')
    k1, k2 = jax.random.split(jax.random.key(123))
    n_i = jax.random.normal(k1, (_B, _H, _W, 1), dtype=jnp.float32)
    n_j = jax.random.normal(k2, (_B, _H, _W, 1), dtype=jnp.float32)
    a_i = ii[None, :, :, None] + n_i * _NOISE
    a_j = jj[None, :, :, None] + n_j * _NOISE
    a = jnp.concatenate([a_i, a_j], axis=3)
    a = jnp.floor(a + 0.4999).astype(jnp.int32)
    a = jnp.where(a < 0, 0, a)
    s = jnp.array([_H - 1, _W - 1], dtype=jnp.int32)
    a = jnp.where(a > s, s, a)
    ai, aj = a[..., 0], a[..., 1]
    return ((ai % _RING) * (_SC // 8) << 8) | aj


def _sc_scramble(table, pidx):
    mesh = plsc.VectorSubcoreMesh(core_axis_name="c", subcore_axis_name="s")

    @functools.partial(
        pl.kernel,
        out_type=jax.ShapeDtypeStruct((_B, _H, _C // 8, 8, _W),
                                       jnp.float32),
        mesh=mesh,
        scratch_types=[
            pltpu.VMEM((_RING * _SC // 8, 8, _W), jnp.float32),
            pltpu.VMEM((2, _SC // 8, 8, _W), jnp.float32),
            pltpu.VMEM((2, 8, _W), jnp.int32),
            pltpu.SemaphoreType.DMA,
            pltpu.SemaphoreType.DMA,
            pltpu.SemaphoreType.DMA,
        ],
        compiler_params=pltpu.CompilerParams(use_tc_tiling_on_sc=True,
                                             needs_layout_passes=False),
    )
    def k(table_hbm, pidx_hbm, out_hbm, band, obuf, pv, lsem, ssem, psem):
        wid = lax.axis_index("s") * 2 + lax.axis_index("c")
        b = wid // _NSLAB
        ct0 = (wid % _NSLAB) * (_SC // 8)
        nct = _SC // 8

        def load_band(r):
            pltpu.async_copy(table_hbm.at[b, r, pl.ds(ct0, nct)],
                             band.at[pl.ds(lax.rem(r, _RING) * nct, nct)],
                             lsem)

        def wait_band():
            pltpu.make_async_copy(table_hbm.at[0, 0, pl.ds(0, nct)],
                                  band.at[pl.ds(0, nct)], lsem).wait()

        def load_pidx(c):
            pltpu.async_copy(pidx_hbm.at[b, pl.ds(c * 8, 8)],
                             pv.at[lax.rem(c, 2)], psem)

        def wait_pidx():
            pltpu.make_async_copy(pidx_hbm.at[0, pl.ds(0, 8)], pv.at[0],
                                  psem).wait()

        def wait_store():
            pltpu.make_async_copy(obuf.at[0], out_hbm.at[0, 0, pl.ds(0, nct)],
                                  ssem).wait()

        for r in range(3):
            load_band(r)
        load_pidx(0)
        for _ in range(3):
            wait_band()
        wait_pidx()

        def row_body(i, _):
            @pl.when(i + 3 <= _H - 1)
            def _():
                load_band(i + 3)

            @pl.when(jnp.logical_and(i >= 1, i + 2 <= _H - 1))
            def _():
                wait_band()

            c = i // 8

            @pl.when(lax.rem(i, 8) == 0)
            def _():
                @pl.when(i > 0)
                def _():
                    wait_pidx()

                @pl.when(c + 1 <= _H // 8 - 1)
                def _():
                    load_pidx(c + 1)

            cb = lax.rem(c, 2)
            r8 = lax.rem(i, 8)
            d = lax.rem(i, 2)

            @pl.when(i >= 2)
            def _():
                wait_store()

            def t_body(t, carry):
                for g in range(_NG):
                    pvec = pv[cb, r8, pl.ds(16 * g, 16)]
                    mb = lax.shift_right_logical(pvec, 8)
                    cv = jnp.bitwise_and(pvec, 255)
                    maj = mb + t
                    for k in range(8):
                        kv = jnp.full((16,), k, jnp.int32)
                        val = plsc.load_gather(band, [maj, kv, cv])
                        obuf[d, t, k, pl.ds(16 * g, 16)] = val
                return carry

            lax.fori_loop(0, nct, t_body, 0)

            pltpu.async_copy(obuf.at[d], out_hbm.at[b, i, pl.ds(ct0, nct)],
                             ssem)
            return 0

        lax.fori_loop(0, _H, row_body, 0)
        wait_store()
        wait_store()

    return k(table, pidx)


def kernel(image):
    table = image.transpose(0, 1, 3, 2).reshape(_B, _H, _C // 8, 8, _W)
    out = _sc_scramble(table, _index_tables())
    return out.reshape(_B, _H, _C, _W).transpose(0, 1, 3, 2)


# speedup vs baseline: 10.1094x; 2.2784x over previous
import functools

import jax
import jax.numpy as jnp
from jax import lax
from jax.experimental import pallas as pl
from jax.experimental.pallas import tpu as pltpu
from jax.experimental.pallas import tpu_sc as plsc

_NOISE = 0.5
_B, _H, _W, _C = 8, 224, 224, 192
_NW = 32
_NSLAB = 4
_SC = _C // _NSLAB
_RING = 6
_NG = _W // 16


def _index_tables():
    ii, jj = jnp.meshgrid(jnp.arange(_H, dtype=jnp.float32),
                          jnp.arange(_W, dtype=jnp.float32), indexing='ij')
    k1, k2 = jax.random.split(jax.random.key(123))
    n_i = jax.random.normal(k1, (_B, _H, _W, 1), dtype=jnp.float32)
    n_j = jax.random.normal(k2, (_B, _H, _W, 1), dtype=jnp.float32)
    a_i = ii[None, :, :, None] + n_i * _NOISE
    a_j = jj[None, :, :, None] + n_j * _NOISE
    a = jnp.concatenate([a_i, a_j], axis=3)
    a = jnp.floor(a + 0.4999).astype(jnp.int32)
    a = jnp.where(a < 0, 0, a)
    s = jnp.array([_H - 1, _W - 1], dtype=jnp.int32)
    a = jnp.where(a > s, s, a)
    ai, aj = a[..., 0], a[..., 1]
    return ((ai % _RING) * (_SC // 8) << 8) | aj


def _sc_scramble(table, pidx):
    mesh = plsc.VectorSubcoreMesh(core_axis_name="c", subcore_axis_name="s")

    @functools.partial(
        pl.kernel,
        out_type=jax.ShapeDtypeStruct((_B, _H, _C // 8, 8, _W),
                                       jnp.float32),
        mesh=mesh,
        scratch_types=[
            pltpu.VMEM((_RING * _SC // 8, 8, _W), jnp.float32),
            pltpu.VMEM((2, _SC // 8, 8, _W), jnp.float32),
            pltpu.VMEM((2, 8, _W), jnp.int32),
            pltpu.SemaphoreType.DMA,
            pltpu.SemaphoreType.DMA,
            pltpu.SemaphoreType.DMA,
        ],
        compiler_params=pltpu.CompilerParams(use_tc_tiling_on_sc=True,
                                             needs_layout_passes=False),
    )
    def k(table_hbm, pidx_hbm, out_hbm, band, obuf, pv, lsem, ssem, psem):
        wid = lax.axis_index("s") * 2 + lax.axis_index("c")
        b = wid // _NSLAB
        ct0 = (wid % _NSLAB) * (_SC // 8)
        nct = _SC // 8

        def load_band(r):
            pltpu.async_copy(table_hbm.at[b, r, pl.ds(ct0, nct)],
                             band.at[pl.ds(lax.rem(r, _RING) * nct, nct)],
                             lsem)

        def wait_band():
            pltpu.make_async_copy(table_hbm.at[0, 0, pl.ds(0, nct)],
                                  band.at[pl.ds(0, nct)], lsem).wait()

        def load_pidx(c):
            pltpu.async_copy(pidx_hbm.at[b, pl.ds(c * 8, 8)],
                             pv.at[lax.rem(c, 2)], psem)

        def wait_pidx():
            pltpu.make_async_copy(pidx_hbm.at[0, pl.ds(0, 8)], pv.at[0],
                                  psem).wait()

        def wait_store():
            pltpu.make_async_copy(obuf.at[0], out_hbm.at[0, 0, pl.ds(0, nct)],
                                  ssem).wait()

        for r in range(3):
            load_band(r)
        load_pidx(0)
        for _ in range(3):
            wait_band()
        wait_pidx()

        def row_body(i, _):
            @pl.when(i + 3 <= _H - 1)
            def _():
                load_band(i + 3)

            @pl.when(jnp.logical_and(i >= 1, i + 2 <= _H - 1))
            def _():
                wait_band()

            c = i // 8

            @pl.when(lax.rem(i, 8) == 0)
            def _():
                @pl.when(i > 0)
                def _():
                    wait_pidx()

                @pl.when(c + 1 <= _H // 8 - 1)
                def _():
                    load_pidx(c + 1)

            cb = lax.rem(c, 2)
            r8 = lax.rem(i, 8)
            d = lax.rem(i, 2)

            @pl.when(i >= 2)
            def _():
                wait_store()

            mbs, cvs = [], []
            for g in range(_NG):
                pvec = pv[cb, r8, pl.ds(16 * g, 16)]
                mbs.append(lax.shift_right_logical(pvec, 8))
                cvs.append(jnp.bitwise_and(pvec, 255))
            for t in range(_SC // 8):
                for g in range(_NG):
                    maj = mbs[g] + t
                    vals = [plsc.load_gather(
                                band, [maj, jnp.full((16,), k, jnp.int32),
                                       cvs[g]])
                            for k in range(8)]
                    for k in range(8):
                        obuf[d, t, k, pl.ds(16 * g, 16)] = vals[k]

            pltpu.async_copy(obuf.at[d], out_hbm.at[b, i, pl.ds(ct0, nct)],
                             ssem)
            return 0

        lax.fori_loop(0, _H, row_body, 0)
        wait_store()
        wait_store()

    return k(table, pidx)


def kernel(image):
    table = image.transpose(0, 1, 3, 2).reshape(_B, _H, _C // 8, 8, _W)
    out = _sc_scramble(table, _index_tables())
    return out.reshape(_B, _H, _C, _W).transpose(0, 1, 3, 2)


# speedup vs baseline: 10.7419x; 1.0626x over previous
import functools

import jax
import jax.numpy as jnp
from jax import lax
from jax.experimental import pallas as pl
from jax.experimental.pallas import tpu as pltpu
from jax.experimental.pallas import tpu_sc as plsc

_NOISE = 0.5
_B, _H, _W, _C = 8, 224, 224, 192
_NW = 32
_NSLAB = 4
_SC = _C // _NSLAB
_RING = 8
_NG = _W // 16


def _index_tables():
    ii, jj = jnp.meshgrid(jnp.arange(_H, dtype=jnp.float32),
                          jnp.arange(_W, dtype=jnp.float32), indexing='ij')
    k1, k2 = jax.random.split(jax.random.key(123))
    n_i = jax.random.normal(k1, (_B, _H, _W, 1), dtype=jnp.float32)
    n_j = jax.random.normal(k2, (_B, _H, _W, 1), dtype=jnp.float32)
    a_i = ii[None, :, :, None] + n_i * _NOISE
    a_j = jj[None, :, :, None] + n_j * _NOISE
    a = jnp.concatenate([a_i, a_j], axis=3)
    a = jnp.floor(a + 0.4999).astype(jnp.int32)
    a = jnp.where(a < 0, 0, a)
    s = jnp.array([_H - 1, _W - 1], dtype=jnp.int32)
    a = jnp.where(a > s, s, a)
    ai, aj = a[..., 0], a[..., 1]
    return ((ai % _RING) * (_SC // 8) << 8) | aj


def _sc_scramble(table, pidx):
    mesh = plsc.VectorSubcoreMesh(core_axis_name="c", subcore_axis_name="s")

    @functools.partial(
        pl.kernel,
        out_type=jax.ShapeDtypeStruct((_B, _H, _C // 8, 8, _W),
                                       jnp.float32),
        mesh=mesh,
        scratch_types=[
            pltpu.VMEM((_RING * _SC // 8, 8, _W), jnp.float32),
            pltpu.VMEM((2, _SC // 8, 8, _W), jnp.float32),
            pltpu.VMEM((2, 8, _W), jnp.int32),
            pltpu.SemaphoreType.DMA,
            pltpu.SemaphoreType.DMA,
            pltpu.SemaphoreType.DMA,
        ],
        compiler_params=pltpu.CompilerParams(use_tc_tiling_on_sc=True,
                                             needs_layout_passes=False),
    )
    def k(table_hbm, pidx_hbm, out_hbm, band, obuf, pv, lsem, ssem, psem):
        wid = lax.axis_index("s") * 2 + lax.axis_index("c")
        b = wid // _NSLAB
        ct0 = (wid % _NSLAB) * (_SC // 8)
        nct = _SC // 8

        def load_band(r):
            pltpu.async_copy(table_hbm.at[b, r, pl.ds(ct0, nct)],
                             band.at[pl.ds(lax.rem(r, _RING) * nct, nct)],
                             lsem)

        def wait_band():
            pltpu.make_async_copy(table_hbm.at[0, 0, pl.ds(0, nct)],
                                  band.at[pl.ds(0, nct)], lsem).wait()

        def load_pidx(c):
            pltpu.async_copy(pidx_hbm.at[b, pl.ds(c * 8, 8)],
                             pv.at[lax.rem(c, 2)], psem)

        def wait_pidx():
            pltpu.make_async_copy(pidx_hbm.at[0, pl.ds(0, 8)], pv.at[0],
                                  psem).wait()

        def wait_store():
            pltpu.make_async_copy(obuf.at[0], out_hbm.at[0, 0, pl.ds(0, nct)],
                                  ssem).wait()

        for r in range(5):
            load_band(r)
        load_pidx(0)
        for _ in range(3):
            wait_band()
        wait_pidx()

        def row_body(i, _):
            @pl.when(i + 5 <= _H - 1)
            def _():
                load_band(i + 5)

            @pl.when(jnp.logical_and(i >= 1, i + 2 <= _H - 1))
            def _():
                wait_band()

            c = i // 8

            @pl.when(lax.rem(i, 8) == 0)
            def _():
                @pl.when(i > 0)
                def _():
                    wait_pidx()

                @pl.when(c + 1 <= _H // 8 - 1)
                def _():
                    load_pidx(c + 1)

            cb = lax.rem(c, 2)
            r8 = lax.rem(i, 8)
            d = lax.rem(i, 2)

            @pl.when(i >= 2)
            def _():
                wait_store()

            mbs, cvs = [], []
            for g in range(_NG):
                pvec = pv[cb, r8, pl.ds(16 * g, 16)]
                mbs.append(lax.shift_right_logical(pvec, 8))
                cvs.append(jnp.bitwise_and(pvec, 255))
            for t in range(_SC // 8):
                for g in range(_NG):
                    maj = mbs[g] + t
                    vals = [plsc.load_gather(
                                band, [maj, jnp.full((16,), k, jnp.int32),
                                       cvs[g]])
                            for k in range(8)]
                    for k in range(8):
                        obuf[d, t, k, pl.ds(16 * g, 16)] = vals[k]

            pltpu.async_copy(obuf.at[d], out_hbm.at[b, i, pl.ds(ct0, nct)],
                             ssem)
            return 0

        lax.fori_loop(0, _H, row_body, 0)
        wait_store()
        wait_store()

    return k(table, pidx)


def kernel(image):
    table = image.transpose(0, 1, 3, 2).reshape(_B, _H, _C // 8, 8, _W)
    out = _sc_scramble(table, _index_tables())
    return out.reshape(_B, _H, _C, _W).transpose(0, 1, 3, 2)


# speedup vs baseline: 10.9635x; 1.0206x over previous
import functools

import jax
import jax.numpy as jnp
import numpy as np
from jax import lax
from jax.experimental import pallas as pl
from jax.experimental.pallas import tpu as pltpu
from jax.experimental.pallas import tpu_sc as plsc

_NOISE = 0.5
_B, _H, _W, _C = 8, 224, 224, 192
_NW = 32
_NSLAB = 4
_SC = _C // _NSLAB
_RING = 8
_NG = _W // 16


def _index_tables():
    ii, jj = jnp.meshgrid(jnp.arange(_H, dtype=jnp.float32),
                          jnp.arange(_W, dtype=jnp.float32), indexing='ij')
    k1, k2 = jax.random.split(jax.random.key(123))
    n_i = jax.random.normal(k1, (_B, _H, _W, 1), dtype=jnp.float32)
    n_j = jax.random.normal(k2, (_B, _H, _W, 1), dtype=jnp.float32)
    a_i = ii[None, :, :, None] + n_i * _NOISE
    a_j = jj[None, :, :, None] + n_j * _NOISE
    a = jnp.concatenate([a_i, a_j], axis=3)
    a = jnp.floor(a + 0.4999).astype(jnp.int32)
    a = jnp.where(a < 0, 0, a)
    s = jnp.array([_H - 1, _W - 1], dtype=jnp.int32)
    a = jnp.where(a > s, s, a)
    ai, aj = a[..., 0], a[..., 1]
    return ((ai % _RING) * (_SC // 8) << 8) | aj


def _sc_scramble(table, pidx):
    mesh = plsc.VectorSubcoreMesh(core_axis_name="c", subcore_axis_name="s")

    @functools.partial(
        pl.kernel,
        out_type=jax.ShapeDtypeStruct((_B, _H, _C // 8, 8, _W),
                                       jnp.float32),
        mesh=mesh,
        scratch_types=[
            pltpu.VMEM((_RING * _SC // 8, 8, _W), jnp.float32),
            pltpu.VMEM((2, _SC // 8, 8, _W), jnp.float32),
            pltpu.VMEM((2, 8, _W), jnp.int32),
            pltpu.SemaphoreType.DMA,
            pltpu.SemaphoreType.DMA,
            pltpu.SemaphoreType.DMA,
        ],
        compiler_params=pltpu.CompilerParams(use_tc_tiling_on_sc=True,
                                             needs_layout_passes=False),
    )
    def k(table_hbm, pidx_hbm, out_hbm, band, obuf, pv, lsem, ssem, psem):
        wid = lax.axis_index("s") * 2 + lax.axis_index("c")
        b = wid // _NSLAB
        ct0 = (wid % _NSLAB) * (_SC // 8)
        nct = _SC // 8

        def load_band(r):
            pltpu.async_copy(table_hbm.at[b, r, pl.ds(ct0, nct)],
                             band.at[pl.ds(lax.rem(r, _RING) * nct, nct)],
                             lsem)

        def wait_band():
            pltpu.make_async_copy(table_hbm.at[0, 0, pl.ds(0, nct)],
                                  band.at[pl.ds(0, nct)], lsem).wait()

        def load_pidx(c):
            pltpu.async_copy(pidx_hbm.at[b, pl.ds(c * 8, 8)],
                             pv.at[lax.rem(c, 2)], psem)

        def wait_pidx():
            pltpu.make_async_copy(pidx_hbm.at[0, pl.ds(0, 8)], pv.at[0],
                                  psem).wait()

        def wait_store():
            pltpu.make_async_copy(obuf.at[0], out_hbm.at[0, 0, pl.ds(0, nct)],
                                  ssem).wait()

        for r in range(5):
            load_band(r)
        load_pidx(0)
        for _ in range(3):
            wait_band()
        wait_pidx()

        def row_body(i, _):
            @pl.when(i + 5 <= _H - 1)
            def _():
                load_band(i + 5)

            @pl.when(jnp.logical_and(i >= 1, i + 2 <= _H - 1))
            def _():
                wait_band()

            c = i // 8

            @pl.when(lax.rem(i, 8) == 0)
            def _():
                @pl.when(i > 0)
                def _():
                    wait_pidx()

                @pl.when(c + 1 <= _H // 8 - 1)
                def _():
                    load_pidx(c + 1)

            cb = lax.rem(c, 2)
            r8 = lax.rem(i, 8)
            d = lax.rem(i, 2)

            @pl.when(i >= 2)
            def _():
                wait_store()

            mbs, cvs = [], []
            for g in range(_NG):
                pvec = pv[cb, r8, pl.ds(16 * g, 16)]
                mbs.append(lax.shift_right_logical(pvec, 8))
                cvs.append(jnp.bitwise_and(pvec, 255))
            for t in range(_SC // 8):
                for g in range(_NG):
                    maj = mbs[g] + t
                    vals = [plsc.load_gather(
                                band, [maj, jnp.full((16,), k, jnp.int32),
                                       cvs[g]])
                            for k in range(8)]
                    for k in range(8):
                        obuf[d, t, k, pl.ds(16 * g, 16)] = vals[k]

            pltpu.async_copy(obuf.at[d], out_hbm.at[b, i, pl.ds(ct0, nct)],
                             ssem)
            return 0

        lax.fori_loop(0, _H, row_body, 0)
        wait_store()
        wait_store()

    return k(table, pidx)


_PIDX_CACHE = []


def _pidx_const():
    if not _PIDX_CACHE:
        try:
            _PIDX_CACHE.append(np.asarray(_index_tables()))
        except Exception:
            return _index_tables()
    return jnp.asarray(_PIDX_CACHE[0])


def kernel(image):
    table = image.transpose(0, 1, 3, 2).reshape(_B, _H, _C // 8, 8, _W)
    out = _sc_scramble(table, _pidx_const())
    return out.reshape(_B, _H, _C, _W).transpose(0, 1, 3, 2)
